# baseline (speedup 1.0000x reference)
import functools

import jax
import jax.numpy as jnp
from jax import lax
from jax.experimental import pallas as pl
from jax.experimental.pallas import tpu as pltpu
from jax.experimental.pallas import tpu_sc as plsc

NC = 2
NS = 16
NW = NC * NS
CHUNK = 128


@functools.lru_cache(maxsize=None)
def _build_sc_gather(B, WIN, V, D):
    n_ctx = B * WIN
    b_per_tile = B // NW
    n_chunks = b_per_tile // CHUNK
    f32 = jnp.float32

    mesh = plsc.VectorSubcoreMesh(
        core_axis_name="c", subcore_axis_name="s", num_cores=NC, num_subcores=NS
    )

    @functools.partial(
        pl.kernel,
        out_type=[
            jax.ShapeDtypeStruct((B // 2, 2 * D), f32),
            jax.ShapeDtypeStruct((B // 2, 2 * D), f32),
            jax.ShapeDtypeStruct((B,), f32),
            jax.ShapeDtypeStruct((n_ctx // 2, 2 * D), f32),
            jax.ShapeDtypeStruct((n_ctx // 2, 2 * D), f32),
            jax.ShapeDtypeStruct((n_ctx,), f32),
            jax.ShapeDtypeStruct((n_ctx // 2, 2 * D), f32),
            jax.ShapeDtypeStruct((n_ctx,), f32),
        ],
        mesh=mesh,
        compiler_params=pltpu.CompilerParams(use_tc_tiling_on_sc=False),
        scratch_types=[
            pltpu.VMEM((CHUNK,), jnp.int32),
            pltpu.VMEM((CHUNK, D), f32),
            pltpu.VMEM((CHUNK, D), f32),
            pltpu.VMEM((CHUNK,), f32),
            pltpu.SemaphoreType.DMA,
            pltpu.SemaphoreType.DMA,
            pltpu.SemaphoreType.DMA,
        ],
    )
    def sc_gather(center_hbm, ctx_hbm, neg_hbm, emb_mu_hbm, lsig_hbm, enc_hbm,
                  enc_c_out, mu_c_out, ls_c_out, enc_ctx_out, mu_ctx_out,
                  ls_ctx_out, mu_neg_out, ls_neg_out,
                  idx_v, rows_enc, rows_mu, ls_v, sem_e, sem_m, sem_l):
        wid = lax.axis_index("s") * NC + lax.axis_index("c")

        H2 = B // 2

        def do_chunk(ids_hbm, base, row, half, enc_out, mu_out, ls_out):
            pltpu.sync_copy(ids_hbm.at[pl.ds(base, CHUNK)], idx_v)
            cp_e = None
            if enc_out is not None:
                cp_e = pltpu.async_copy(enc_hbm.at[idx_v], rows_enc, sem_e)
            cp_m = pltpu.async_copy(emb_mu_hbm.at[idx_v], rows_mu, sem_m)
            cp_l = pltpu.async_copy(lsig_hbm.at[idx_v], ls_v, sem_l)
            if cp_e is not None:
                cp_e.wait()
                pltpu.sync_copy(
                    rows_enc,
                    enc_out.at[pl.ds(row, CHUNK), pl.ds(half * D, D)])
            cp_m.wait()
            pltpu.sync_copy(
                rows_mu, mu_out.at[pl.ds(row, CHUNK), pl.ds(half * D, D)])
            cp_l.wait()
            pltpu.sync_copy(ls_v, ls_out.at[pl.ds(base, CHUNK)])

        b0 = wid * b_per_tile

        @pl.loop(0, n_chunks)
        def _center(k):
            off = b0 + k * CHUNK
            do_chunk(center_hbm, off, off % H2, off // H2,
                     enc_c_out, mu_c_out, ls_c_out)

        @pl.loop(0, WIN * n_chunks)
        def _ctx(k):
            w = k // n_chunks
            c = k % n_chunks
            off = b0 + c * CHUNK
            do_chunk(ctx_hbm, w * B + off, w * H2 + off % H2, off // H2,
                     enc_ctx_out, mu_ctx_out, ls_ctx_out)

        @pl.loop(0, WIN * n_chunks)
        def _neg(k):
            w = k // n_chunks
            c = k % n_chunks
            off = b0 + c * CHUNK
            do_chunk(neg_hbm, w * B + off, w * H2 + off % H2, off // H2,
                     None, mu_neg_out, ls_neg_out)

    return sc_gather


def _tc_body(WIN, LATENT, H,
             enc_c_ref, enc_ctx_ref, mu_c_ref, mu_ctx_ref, mu_neg_ref,
             ls_c2_ref, ls_ctx_ref, ls_neg_ref,
             wfc_ref, wfx_ref, bf_ref, wmu_ref, bmu_ref, wsig_ref, bsig_ref,
             bdp_ref, bd3_ref, t2_ref, kl_ref, hinge_ref):
    f32 = jnp.float32
    d = float(LATENT)
    W2 = 2 * WIN
    ecp = enc_c_ref[...]
    hc = jnp.dot(ecp, wfc_ref[...], preferred_element_type=f32) + bf_ref[...]
    h = jnp.zeros(hc.shape, f32)
    wfx = wfx_ref[...]
    for w in range(WIN):
        ew = enc_ctx_ref[w][...]
        hw = jnp.dot(ew, wfx, preferred_element_type=f32)
        h = h + jnp.maximum(hc + hw, 0.0)
    mu_q2p = (jnp.dot(h, wmu_ref[...], preferred_element_type=f32)
              + bmu_ref[...])
    sg2 = (jnp.dot(h, wsig_ref[...], preferred_element_type=f32)
           + bsig_ref[...])
    sigma2 = (jnp.maximum(sg2, 0.0)
              + jnp.log(1.0 + jnp.exp(-jnp.abs(sg2))) + 1e-6)
    lsq2 = jnp.log(sigma2)
    aq2 = d * sigma2 * sigma2

    def kl(sq, ls_t, lsq_x, aq_x):
        return (d * (ls_t - lsq_x)
                + (aq_x + sq) * (0.5 * jnp.exp(-2.0 * ls_t)) - 0.5 * d)

    dc = mu_q2p - mu_c_ref[...]
    sqc = jnp.dot(dc * dc, bdp_ref[...], preferred_element_type=f32)
    kl_part = jnp.sum(kl(sqc, ls_c2_ref[...], lsq2, aq2), keepdims=True)

    t2 = t2_ref[...]
    lsq40 = jnp.dot(lsq2, t2, preferred_element_type=f32)
    aq40 = jnp.dot(aq2, t2, preferred_element_type=f32)
    sqp = jnp.zeros((sqc.shape[0], W2), f32)
    sqn = jnp.zeros((sqc.shape[0], W2), f32)
    for w in range(WIN):
        bdw = bd3_ref[w][...]
        dp = mu_q2p - mu_ctx_ref[w][...]
        dn = mu_q2p - mu_neg_ref[w][...]
        sqp = sqp + jnp.dot(dp * dp, bdw, preferred_element_type=f32)
        sqn = sqn + jnp.dot(dn * dn, bdw, preferred_element_type=f32)
    klp = kl(sqp, ls_ctx_ref[...], lsq40, aq40)
    kln = kl(sqn, ls_neg_ref[...], lsq40, aq40)
    hinge_part = jnp.sum(jnp.maximum(klp - kln + 1.0, 0.0), keepdims=True)

    i = pl.program_id(0)

    @pl.when(i == 0)
    def _():
        kl_ref[...] = kl_part
        hinge_ref[...] = hinge_part

    @pl.when(i > 0)
    def _():
        kl_ref[...] += kl_part
        hinge_ref[...] += hinge_part


@functools.lru_cache(maxsize=None)
def _build_tc(B, WIN, D, LATENT, H, Bb):
    f32 = jnp.float32
    grid = (B // Bb,)
    body = functools.partial(_tc_body, WIN, LATENT, H)
    const = lambda *shape: pl.BlockSpec(shape, lambda i: (0,) * len(shape))
    return pl.pallas_call(
        body,
        grid=grid,
        in_specs=[
            pl.BlockSpec((Bb // 2, 2 * D), lambda i: (i, 0)),
            pl.BlockSpec((WIN, Bb // 2, 2 * D), lambda i: (0, i, 0)),
            pl.BlockSpec((Bb // 2, 2 * LATENT), lambda i: (i, 0)),
            pl.BlockSpec((WIN, Bb // 2, 2 * LATENT), lambda i: (0, i, 0)),
            pl.BlockSpec((WIN, Bb // 2, 2 * LATENT), lambda i: (0, i, 0)),
            pl.BlockSpec((Bb // 2, 2), lambda i: (i, 0)),
            pl.BlockSpec((Bb // 2, 2 * WIN), lambda i: (i, 0)),
            pl.BlockSpec((Bb // 2, 2 * WIN), lambda i: (i, 0)),
            const(2 * D, 2 * H),
            const(2 * D, 2 * H),
            const(1, 2 * H),
            const(2 * H, 2 * LATENT),
            const(1, 2 * LATENT),
            const(2 * H, 2),
            const(1, 2),
            const(2 * LATENT, 2),
            const(WIN, 2 * LATENT, 2 * WIN),
            const(2, 2 * WIN),
        ],
        out_specs=[const(1, 1), const(1, 1)],
        out_shape=[
            jax.ShapeDtypeStruct((1, 1), f32),
            jax.ShapeDtypeStruct((1, 1), f32),
        ],
    )


def kernel(center_ids, context_ids, neg_context_ids, emb_mu, emb_log_sigma,
           enc_emb, W_f, b_f, W_mu, b_mu, W_sig, b_sig):
    B = center_ids.shape[0]
    WIN = context_ids.shape[1]
    V, LATENT = emb_mu.shape
    D = enc_emb.shape[1]
    H = W_f.shape[1]

    c_ids = center_ids.astype(jnp.int32)
    ctx_ids = context_ids.astype(jnp.int32)
    neg_ids = neg_context_ids.astype(jnp.int32)
    lsig = emb_log_sigma.reshape(-1)

    NSPLIT = 2
    B2 = B // NSPLIT
    sc = _build_sc_gather(B2, WIN, V, D)

    def il40(x):
        return (x.reshape(WIN, 2, B2 // 2).transpose(2, 0, 1)
                 .reshape(B2 // 2, 2 * WIN))

    j2 = jnp.arange(2 * LATENT, dtype=jnp.int32)
    cols = jnp.arange(2 * WIN, dtype=jnp.int32)
    bd3 = (cols[None, None, :] ==
           (2 * jnp.arange(WIN, dtype=jnp.int32)[:, None, None]
            + j2[None, :, None] // LATENT)).astype(jnp.float32)
    bdp = (j2[:, None] // LATENT ==
           jnp.arange(2, dtype=jnp.int32)[None, :]).astype(jnp.float32)
    t2 = (cols[None, :] % 2 ==
          jnp.arange(2, dtype=jnp.int32)[:, None]).astype(jnp.float32)

    def blkdiag(a):
        m, n = a.shape
        z = jnp.zeros((m, n), a.dtype)
        return jnp.concatenate(
            [jnp.concatenate([a, z], axis=1),
             jnp.concatenate([z, a], axis=1)], axis=0)

    wfc = blkdiag(W_f[:D])
    wfx = blkdiag(W_f[D:])
    wmu2 = blkdiag(W_mu)
    wsig2 = blkdiag(W_sig.reshape(H, 1))
    bf2 = jnp.concatenate([b_f.reshape(1, H)] * 2, axis=1)
    bmu2 = jnp.concatenate([b_mu.reshape(1, LATENT)] * 2, axis=1)
    bsig2 = jnp.concatenate([b_sig.reshape(1, 1)] * 2, axis=1)

    Bb = 512
    tc = _build_tc(B2, WIN, D, LATENT, H, Bb)
    kl_sum = jnp.zeros((), jnp.float32)
    hinge_sum = jnp.zeros((), jnp.float32)
    for hs in range(NSPLIT):
        s = slice(hs * B2, (hs + 1) * B2)
        (enc_c, mu_c, ls_c, enc_ctx, mu_ctx, ls_ctx, mu_neg, ls_neg) = sc(
            c_ids[s], ctx_ids[s].T.reshape(-1), neg_ids[s].T.reshape(-1),
            emb_mu, lsig, enc_emb)
        kl_h, hinge_h = tc(
            enc_c, enc_ctx.reshape(WIN, B2 // 2, 2 * D), mu_c,
            mu_ctx.reshape(WIN, B2 // 2, 2 * LATENT),
            mu_neg.reshape(WIN, B2 // 2, 2 * LATENT),
            ls_c.reshape(2, B2 // 2).transpose(1, 0),
            il40(ls_ctx), il40(ls_neg),
            wfc, wfx, bf2, wmu2, bmu2, wsig2, bsig2, bdp, bd3, t2)
        kl_sum = kl_sum + kl_h[0, 0]
        hinge_sum = hinge_sum + hinge_h[0, 0]

    kl = kl_sum / B
    max_margin = hinge_sum / (B * WIN)
    return (kl, max_margin)


# speedup vs baseline: 1.2414x; 1.2414x over previous
import functools

import jax
import jax.numpy as jnp
from jax import lax
from jax.experimental import pallas as pl
from jax.experimental.pallas import tpu as pltpu
from jax.experimental.pallas import tpu_sc as plsc

NC = 2
NS = 16
NW = NC * NS
CHUNK = 128


@functools.lru_cache(maxsize=None)
def _build_sc_gather(B, WIN, V, D):
    n_ctx = B * WIN
    b_per_tile = B // NW
    n_chunks = b_per_tile // CHUNK
    f32 = jnp.float32

    mesh = plsc.VectorSubcoreMesh(
        core_axis_name="c", subcore_axis_name="s", num_cores=NC, num_subcores=NS
    )

    @functools.partial(
        pl.kernel,
        out_type=[
            jax.ShapeDtypeStruct((B // 2, 2 * D), f32),
            jax.ShapeDtypeStruct((B // 2, 2 * D), f32),
            jax.ShapeDtypeStruct((B,), f32),
            jax.ShapeDtypeStruct((n_ctx // 2, 2 * D), f32),
            jax.ShapeDtypeStruct((n_ctx // 2, 2 * D), f32),
            jax.ShapeDtypeStruct((n_ctx,), f32),
            jax.ShapeDtypeStruct((n_ctx // 2, 2 * D), f32),
            jax.ShapeDtypeStruct((n_ctx,), f32),
        ],
        mesh=mesh,
        compiler_params=pltpu.CompilerParams(use_tc_tiling_on_sc=False),
        scratch_types=[
            pltpu.VMEM((2, CHUNK), jnp.int32),
            pltpu.VMEM((2, CHUNK, D), f32),
            pltpu.VMEM((2, CHUNK, D), f32),
            pltpu.VMEM((2, CHUNK), f32),
            pltpu.SemaphoreType.DMA,
            pltpu.SemaphoreType.DMA,
            pltpu.SemaphoreType.DMA,
            pltpu.SemaphoreType.DMA,
            pltpu.SemaphoreType.DMA,
            pltpu.SemaphoreType.DMA,
        ],
    )
    def sc_gather(center_hbm, ctx_hbm, neg_hbm, emb_mu_hbm, lsig_hbm, enc_hbm,
                  enc_c_out, mu_c_out, ls_c_out, enc_ctx_out, mu_ctx_out,
                  ls_ctx_out, mu_neg_out, ls_neg_out,
                  idx_v, rows_enc, rows_mu, ls_v,
                  sem_e0, sem_m0, sem_l0, sem_e1, sem_m1, sem_l1):
        wid = lax.axis_index("s") * NC + lax.axis_index("c")

        H2 = B // 2
        sems = ((sem_e0, sem_m0, sem_l0), (sem_e1, sem_m1, sem_l1))

        def fire(buf, ids_hbm, base, do_enc):
            pltpu.sync_copy(ids_hbm.at[pl.ds(base, CHUNK)], idx_v.at[buf])
            se, sm, sl = sems[buf]
            cp_e = None
            if do_enc:
                cp_e = pltpu.async_copy(
                    enc_hbm.at[idx_v.at[buf]], rows_enc.at[buf], se)
            cp_m = pltpu.async_copy(
                emb_mu_hbm.at[idx_v.at[buf]], rows_mu.at[buf], sm)
            cp_l = pltpu.async_copy(
                lsig_hbm.at[idx_v.at[buf]], ls_v.at[buf], sl)
            return cp_e, cp_m, cp_l

        def drain(buf, cps, base, row, half, enc_out, mu_out, ls_out):
            cp_e, cp_m, cp_l = cps
            if cp_e is not None:
                cp_e.wait()
                pltpu.sync_copy(
                    rows_enc.at[buf],
                    enc_out.at[pl.ds(row, CHUNK), pl.ds(half * D, D)])
            cp_m.wait()
            pltpu.sync_copy(
                rows_mu.at[buf],
                mu_out.at[pl.ds(row, CHUNK), pl.ds(half * D, D)])
            cp_l.wait()
            pltpu.sync_copy(ls_v.at[buf], ls_out.at[pl.ds(base, CHUNK)])

        def do_pair(ids_hbm, base0, row0, base1, row1, half,
                    enc_out, mu_out, ls_out):
            cps0 = fire(0, ids_hbm, base0, enc_out is not None)
            cps1 = fire(1, ids_hbm, base1, enc_out is not None)
            drain(0, cps0, base0, row0, half, enc_out, mu_out, ls_out)
            drain(1, cps1, base1, row1, half, enc_out, mu_out, ls_out)

        b0 = wid * b_per_tile
        np2 = n_chunks // 2

        @pl.loop(0, np2)
        def _center(k):
            off = b0 + 2 * k * CHUNK
            do_pair(center_hbm, off, off % H2, off + CHUNK, off % H2 + CHUNK,
                    off // H2, enc_c_out, mu_c_out, ls_c_out)

        @pl.loop(0, WIN * np2)
        def _ctx(k):
            w = k // np2
            c = 2 * (k % np2)
            off = b0 + c * CHUNK
            do_pair(ctx_hbm, w * B + off, w * H2 + off % H2,
                    w * B + off + CHUNK, w * H2 + off % H2 + CHUNK,
                    off // H2, enc_ctx_out, mu_ctx_out, ls_ctx_out)

        @pl.loop(0, WIN * np2)
        def _neg(k):
            w = k // np2
            c = 2 * (k % np2)
            off = b0 + c * CHUNK
            do_pair(neg_hbm, w * B + off, w * H2 + off % H2,
                    w * B + off + CHUNK, w * H2 + off % H2 + CHUNK,
                    off // H2, None, mu_neg_out, ls_neg_out)

    return sc_gather


def _tc_body(WIN, LATENT, H,
             enc_c_ref, enc_ctx_ref, mu_c_ref, mu_ctx_ref, mu_neg_ref,
             ls_c2_ref, ls_ctx_ref, ls_neg_ref,
             wfc_ref, wfx_ref, bf_ref, wmu_ref, bmu_ref, wsig_ref, bsig_ref,
             bdp_ref, bd3_ref, t2_ref, kl_ref, hinge_ref):
    f32 = jnp.float32
    d = float(LATENT)
    W2 = 2 * WIN
    ecp = enc_c_ref[...]
    hc = jnp.dot(ecp, wfc_ref[...], preferred_element_type=f32) + bf_ref[...]
    h = jnp.zeros(hc.shape, f32)
    wfx = wfx_ref[...]
    for w in range(WIN):
        ew = enc_ctx_ref[w][...]
        hw = jnp.dot(ew, wfx, preferred_element_type=f32)
        h = h + jnp.maximum(hc + hw, 0.0)
    mu_q2p = (jnp.dot(h, wmu_ref[...], preferred_element_type=f32)
              + bmu_ref[...])
    sg2 = (jnp.dot(h, wsig_ref[...], preferred_element_type=f32)
           + bsig_ref[...])
    sigma2 = (jnp.maximum(sg2, 0.0)
              + jnp.log(1.0 + jnp.exp(-jnp.abs(sg2))) + 1e-6)
    lsq2 = jnp.log(sigma2)
    aq2 = d * sigma2 * sigma2

    def kl(sq, ls_t, lsq_x, aq_x):
        return (d * (ls_t - lsq_x)
                + (aq_x + sq) * (0.5 * jnp.exp(-2.0 * ls_t)) - 0.5 * d)

    dc = mu_q2p - mu_c_ref[...]
    sqc = jnp.dot(dc * dc, bdp_ref[...], preferred_element_type=f32)
    kl_part = jnp.sum(kl(sqc, ls_c2_ref[...], lsq2, aq2), keepdims=True)

    t2 = t2_ref[...]
    lsq40 = jnp.dot(lsq2, t2, preferred_element_type=f32)
    aq40 = jnp.dot(aq2, t2, preferred_element_type=f32)
    sqp = jnp.zeros((sqc.shape[0], W2), f32)
    sqn = jnp.zeros((sqc.shape[0], W2), f32)
    for w in range(WIN):
        bdw = bd3_ref[w][...]
        dp = mu_q2p - mu_ctx_ref[w][...]
        dn = mu_q2p - mu_neg_ref[w][...]
        sqp = sqp + jnp.dot(dp * dp, bdw, preferred_element_type=f32)
        sqn = sqn + jnp.dot(dn * dn, bdw, preferred_element_type=f32)
    klp = kl(sqp, ls_ctx_ref[...], lsq40, aq40)
    kln = kl(sqn, ls_neg_ref[...], lsq40, aq40)
    hinge_part = jnp.sum(jnp.maximum(klp - kln + 1.0, 0.0), keepdims=True)

    i = pl.program_id(0)

    @pl.when(i == 0)
    def _():
        kl_ref[...] = kl_part
        hinge_ref[...] = hinge_part

    @pl.when(i > 0)
    def _():
        kl_ref[...] += kl_part
        hinge_ref[...] += hinge_part


@functools.lru_cache(maxsize=None)
def _build_tc(B, WIN, D, LATENT, H, Bb):
    f32 = jnp.float32
    grid = (B // Bb,)
    body = functools.partial(_tc_body, WIN, LATENT, H)
    const = lambda *shape: pl.BlockSpec(shape, lambda i: (0,) * len(shape))
    return pl.pallas_call(
        body,
        grid=grid,
        in_specs=[
            pl.BlockSpec((Bb // 2, 2 * D), lambda i: (i, 0)),
            pl.BlockSpec((WIN, Bb // 2, 2 * D), lambda i: (0, i, 0)),
            pl.BlockSpec((Bb // 2, 2 * LATENT), lambda i: (i, 0)),
            pl.BlockSpec((WIN, Bb // 2, 2 * LATENT), lambda i: (0, i, 0)),
            pl.BlockSpec((WIN, Bb // 2, 2 * LATENT), lambda i: (0, i, 0)),
            pl.BlockSpec((Bb // 2, 2), lambda i: (i, 0)),
            pl.BlockSpec((Bb // 2, 2 * WIN), lambda i: (i, 0)),
            pl.BlockSpec((Bb // 2, 2 * WIN), lambda i: (i, 0)),
            const(2 * D, 2 * H),
            const(2 * D, 2 * H),
            const(1, 2 * H),
            const(2 * H, 2 * LATENT),
            const(1, 2 * LATENT),
            const(2 * H, 2),
            const(1, 2),
            const(2 * LATENT, 2),
            const(WIN, 2 * LATENT, 2 * WIN),
            const(2, 2 * WIN),
        ],
        out_specs=[const(1, 1), const(1, 1)],
        out_shape=[
            jax.ShapeDtypeStruct((1, 1), f32),
            jax.ShapeDtypeStruct((1, 1), f32),
        ],
    )


def kernel(center_ids, context_ids, neg_context_ids, emb_mu, emb_log_sigma,
           enc_emb, W_f, b_f, W_mu, b_mu, W_sig, b_sig):
    B = center_ids.shape[0]
    WIN = context_ids.shape[1]
    V, LATENT = emb_mu.shape
    D = enc_emb.shape[1]
    H = W_f.shape[1]

    c_ids = center_ids.astype(jnp.int32)
    ctx_ids = context_ids.astype(jnp.int32).T.reshape(-1)
    neg_ids = neg_context_ids.astype(jnp.int32).T.reshape(-1)
    lsig = emb_log_sigma.reshape(-1)

    sc = _build_sc_gather(B, WIN, V, D)
    (enc_c, mu_c, ls_c, enc_ctx, mu_ctx, ls_ctx, mu_neg, ls_neg) = sc(
        c_ids, ctx_ids, neg_ids, emb_mu, lsig, enc_emb)

    def il40(x):
        return (x.reshape(WIN, 2, B // 2).transpose(2, 0, 1)
                 .reshape(B // 2, 2 * WIN))

    j2 = jnp.arange(2 * LATENT, dtype=jnp.int32)
    cols = jnp.arange(2 * WIN, dtype=jnp.int32)
    bd3 = (cols[None, None, :] ==
           (2 * jnp.arange(WIN, dtype=jnp.int32)[:, None, None]
            + j2[None, :, None] // LATENT)).astype(jnp.float32)
    bdp = (j2[:, None] // LATENT ==
           jnp.arange(2, dtype=jnp.int32)[None, :]).astype(jnp.float32)
    t2 = (cols[None, :] % 2 ==
          jnp.arange(2, dtype=jnp.int32)[:, None]).astype(jnp.float32)

    def blkdiag(a):
        m, n = a.shape
        z = jnp.zeros((m, n), a.dtype)
        return jnp.concatenate(
            [jnp.concatenate([a, z], axis=1),
             jnp.concatenate([z, a], axis=1)], axis=0)

    wfc = blkdiag(W_f[:D])
    wfx = blkdiag(W_f[D:])
    wmu2 = blkdiag(W_mu)
    wsig2 = blkdiag(W_sig.reshape(H, 1))
    bf2 = jnp.concatenate([b_f.reshape(1, H)] * 2, axis=1)
    bmu2 = jnp.concatenate([b_mu.reshape(1, LATENT)] * 2, axis=1)
    bsig2 = jnp.concatenate([b_sig.reshape(1, 1)] * 2, axis=1)

    Bb = 512
    tc = _build_tc(B, WIN, D, LATENT, H, Bb)
    kl_sum, hinge_sum = tc(
        enc_c, enc_ctx.reshape(WIN, B // 2, 2 * D), mu_c,
        mu_ctx.reshape(WIN, B // 2, 2 * LATENT),
        mu_neg.reshape(WIN, B // 2, 2 * LATENT),
        ls_c.reshape(2, B // 2).transpose(1, 0), il40(ls_ctx), il40(ls_neg),
        wfc, wfx, bf2, wmu2, bmu2, wsig2, bsig2, bdp, bd3, t2)

    kl = kl_sum[0, 0] / B
    max_margin = hinge_sum[0, 0] / (B * WIN)
    return (kl, max_margin)


# speedup vs baseline: 1.3810x; 1.1124x over previous
import functools

import jax
import jax.numpy as jnp
from jax import lax
from jax.experimental import pallas as pl
from jax.experimental.pallas import tpu as pltpu
from jax.experimental.pallas import tpu_sc as plsc

NC = 2
NS = 16
NW = NC * NS
CHUNK = 256


@functools.lru_cache(maxsize=None)
def _build_sc_gather(B, WIN, V, D):
    n_ctx = B * WIN
    b_per_tile = B // NW
    n_chunks = b_per_tile // CHUNK
    f32 = jnp.float32

    mesh = plsc.VectorSubcoreMesh(
        core_axis_name="c", subcore_axis_name="s", num_cores=NC, num_subcores=NS
    )

    @functools.partial(
        pl.kernel,
        out_type=[
            jax.ShapeDtypeStruct((B // 2, 2 * D), f32),
            jax.ShapeDtypeStruct((B // 2, 2 * D), f32),
            jax.ShapeDtypeStruct((B,), f32),
            jax.ShapeDtypeStruct((n_ctx // 2, 2 * D), f32),
            jax.ShapeDtypeStruct((n_ctx // 2, 2 * D), f32),
            jax.ShapeDtypeStruct((n_ctx,), f32),
            jax.ShapeDtypeStruct((n_ctx // 2, 2 * D), f32),
            jax.ShapeDtypeStruct((n_ctx,), f32),
        ],
        mesh=mesh,
        compiler_params=pltpu.CompilerParams(use_tc_tiling_on_sc=False),
        scratch_types=[
            pltpu.VMEM((2, CHUNK), jnp.int32),
            pltpu.VMEM((2, CHUNK, D), f32),
            pltpu.VMEM((2, CHUNK, D), f32),
            pltpu.VMEM((2, CHUNK), f32),
            pltpu.SemaphoreType.DMA,
            pltpu.SemaphoreType.DMA,
            pltpu.SemaphoreType.DMA,
            pltpu.SemaphoreType.DMA,
            pltpu.SemaphoreType.DMA,
            pltpu.SemaphoreType.DMA,
        ],
    )
    def sc_gather(center_hbm, ctx_hbm, neg_hbm, emb_mu_hbm, lsig_hbm, enc_hbm,
                  enc_c_out, mu_c_out, ls_c_out, enc_ctx_out, mu_ctx_out,
                  ls_ctx_out, mu_neg_out, ls_neg_out,
                  idx_v, rows_enc, rows_mu, ls_v,
                  sem_e0, sem_m0, sem_l0, sem_e1, sem_m1, sem_l1):
        wid = lax.axis_index("s") * NC + lax.axis_index("c")

        H2 = B // 2
        sems = ((sem_e0, sem_m0, sem_l0), (sem_e1, sem_m1, sem_l1))

        def fire(buf, ids_hbm, base, do_enc):
            pltpu.sync_copy(ids_hbm.at[pl.ds(base, CHUNK)], idx_v.at[buf])
            se, sm, sl = sems[buf]
            cp_e = None
            if do_enc:
                cp_e = pltpu.async_copy(
                    enc_hbm.at[idx_v.at[buf]], rows_enc.at[buf], se)
            cp_m = pltpu.async_copy(
                emb_mu_hbm.at[idx_v.at[buf]], rows_mu.at[buf], sm)
            cp_l = pltpu.async_copy(
                lsig_hbm.at[idx_v.at[buf]], ls_v.at[buf], sl)
            return cp_e, cp_m, cp_l

        def drain(buf, cps, base, row, half, enc_out, mu_out, ls_out):
            cp_e, cp_m, cp_l = cps
            if cp_e is not None:
                cp_e.wait()
                pltpu.sync_copy(
                    rows_enc.at[buf],
                    enc_out.at[pl.ds(row, CHUNK), pl.ds(half * D, D)])
            cp_m.wait()
            pltpu.sync_copy(
                rows_mu.at[buf],
                mu_out.at[pl.ds(row, CHUNK), pl.ds(half * D, D)])
            cp_l.wait()
            pltpu.sync_copy(ls_v.at[buf], ls_out.at[pl.ds(base, CHUNK)])

        def do_pair(ids_hbm, base0, row0, base1, row1, half,
                    enc_out, mu_out, ls_out):
            cps0 = fire(0, ids_hbm, base0, enc_out is not None)
            cps1 = fire(1, ids_hbm, base1, enc_out is not None)
            drain(0, cps0, base0, row0, half, enc_out, mu_out, ls_out)
            drain(1, cps1, base1, row1, half, enc_out, mu_out, ls_out)

        b0 = wid * b_per_tile
        np2 = n_chunks // 2

        @pl.loop(0, np2)
        def _center(k):
            off = b0 + 2 * k * CHUNK
            do_pair(center_hbm, off, off % H2, off + CHUNK, off % H2 + CHUNK,
                    off // H2, enc_c_out, mu_c_out, ls_c_out)

        @pl.loop(0, WIN * np2)
        def _ctx(k):
            w = k // np2
            c = 2 * (k % np2)
            off = b0 + c * CHUNK
            do_pair(ctx_hbm, w * B + off, w * H2 + off % H2,
                    w * B + off + CHUNK, w * H2 + off % H2 + CHUNK,
                    off // H2, enc_ctx_out, mu_ctx_out, ls_ctx_out)

        @pl.loop(0, WIN * np2)
        def _neg(k):
            w = k // np2
            c = 2 * (k % np2)
            off = b0 + c * CHUNK
            do_pair(neg_hbm, w * B + off, w * H2 + off % H2,
                    w * B + off + CHUNK, w * H2 + off % H2 + CHUNK,
                    off // H2, None, mu_neg_out, ls_neg_out)

    return sc_gather


def _tc_body(WIN, LATENT, H,
             enc_c_ref, enc_ctx_ref, mu_c_ref, mu_ctx_ref, mu_neg_ref,
             ls_c2_ref, ls_ctx_ref, ls_neg_ref,
             wfc_ref, wfx_ref, bf_ref, wmu_ref, bmu_ref, wsig_ref, bsig_ref,
             bdp_ref, bd3_ref, t2_ref, kl_ref, hinge_ref):
    f32 = jnp.float32
    d = float(LATENT)
    W2 = 2 * WIN
    ecp = enc_c_ref[...]
    hc = jnp.dot(ecp, wfc_ref[...], preferred_element_type=f32) + bf_ref[...]
    h = jnp.zeros(hc.shape, f32)
    wfx = wfx_ref[...]
    for w in range(WIN):
        ew = enc_ctx_ref[w][...]
        hw = jnp.dot(ew, wfx, preferred_element_type=f32)
        h = h + jnp.maximum(hc + hw, 0.0)
    mu_q2p = (jnp.dot(h, wmu_ref[...], preferred_element_type=f32)
              + bmu_ref[...])
    sg2 = (jnp.dot(h, wsig_ref[...], preferred_element_type=f32)
           + bsig_ref[...])
    sigma2 = (jnp.maximum(sg2, 0.0)
              + jnp.log(1.0 + jnp.exp(-jnp.abs(sg2))) + 1e-6)
    lsq2 = jnp.log(sigma2)
    aq2 = d * sigma2 * sigma2

    def kl(sq, ls_t, lsq_x, aq_x):
        return (d * (ls_t - lsq_x)
                + (aq_x + sq) * (0.5 * jnp.exp(-2.0 * ls_t)) - 0.5 * d)

    dc = mu_q2p - mu_c_ref[...]
    sqc = jnp.dot(dc * dc, bdp_ref[...], preferred_element_type=f32)
    kl_part = jnp.sum(kl(sqc, ls_c2_ref[...], lsq2, aq2), keepdims=True)

    t2 = t2_ref[...]
    lsq40 = jnp.dot(lsq2, t2, preferred_element_type=f32)
    aq40 = jnp.dot(aq2, t2, preferred_element_type=f32)
    sqp = jnp.zeros((sqc.shape[0], W2), f32)
    sqn = jnp.zeros((sqc.shape[0], W2), f32)
    for w in range(WIN):
        bdw = bd3_ref[w][...]
        dp = mu_q2p - mu_ctx_ref[w][...]
        dn = mu_q2p - mu_neg_ref[w][...]
        sqp = sqp + jnp.dot(dp * dp, bdw, preferred_element_type=f32)
        sqn = sqn + jnp.dot(dn * dn, bdw, preferred_element_type=f32)
    klp = kl(sqp, ls_ctx_ref[...], lsq40, aq40)
    kln = kl(sqn, ls_neg_ref[...], lsq40, aq40)
    hinge_part = jnp.sum(jnp.maximum(klp - kln + 1.0, 0.0), keepdims=True)

    i = pl.program_id(0)

    @pl.when(i == 0)
    def _():
        kl_ref[...] = kl_part
        hinge_ref[...] = hinge_part

    @pl.when(i > 0)
    def _():
        kl_ref[...] += kl_part
        hinge_ref[...] += hinge_part


@functools.lru_cache(maxsize=None)
def _build_tc(B, WIN, D, LATENT, H, Bb):
    f32 = jnp.float32
    grid = (B // Bb,)
    body = functools.partial(_tc_body, WIN, LATENT, H)
    const = lambda *shape: pl.BlockSpec(shape, lambda i: (0,) * len(shape))
    return pl.pallas_call(
        body,
        grid=grid,
        in_specs=[
            pl.BlockSpec((Bb // 2, 2 * D), lambda i: (i, 0)),
            pl.BlockSpec((WIN, Bb // 2, 2 * D), lambda i: (0, i, 0)),
            pl.BlockSpec((Bb // 2, 2 * LATENT), lambda i: (i, 0)),
            pl.BlockSpec((WIN, Bb // 2, 2 * LATENT), lambda i: (0, i, 0)),
            pl.BlockSpec((WIN, Bb // 2, 2 * LATENT), lambda i: (0, i, 0)),
            pl.BlockSpec((Bb // 2, 2), lambda i: (i, 0)),
            pl.BlockSpec((Bb // 2, 2 * WIN), lambda i: (i, 0)),
            pl.BlockSpec((Bb // 2, 2 * WIN), lambda i: (i, 0)),
            const(2 * D, 2 * H),
            const(2 * D, 2 * H),
            const(1, 2 * H),
            const(2 * H, 2 * LATENT),
            const(1, 2 * LATENT),
            const(2 * H, 2),
            const(1, 2),
            const(2 * LATENT, 2),
            const(WIN, 2 * LATENT, 2 * WIN),
            const(2, 2 * WIN),
        ],
        out_specs=[const(1, 1), const(1, 1)],
        out_shape=[
            jax.ShapeDtypeStruct((1, 1), f32),
            jax.ShapeDtypeStruct((1, 1), f32),
        ],
    )


def kernel(center_ids, context_ids, neg_context_ids, emb_mu, emb_log_sigma,
           enc_emb, W_f, b_f, W_mu, b_mu, W_sig, b_sig):
    B = center_ids.shape[0]
    WIN = context_ids.shape[1]
    V, LATENT = emb_mu.shape
    D = enc_emb.shape[1]
    H = W_f.shape[1]

    c_ids = center_ids.astype(jnp.int32)
    ctx_ids = context_ids.astype(jnp.int32).T.reshape(-1)
    neg_ids = neg_context_ids.astype(jnp.int32).T.reshape(-1)
    lsig = emb_log_sigma.reshape(-1)

    sc = _build_sc_gather(B, WIN, V, D)
    (enc_c, mu_c, ls_c, enc_ctx, mu_ctx, ls_ctx, mu_neg, ls_neg) = sc(
        c_ids, ctx_ids, neg_ids, emb_mu, lsig, enc_emb)

    def il40(x):
        return (x.reshape(WIN, 2, B // 2).transpose(2, 0, 1)
                 .reshape(B // 2, 2 * WIN))

    j2 = jnp.arange(2 * LATENT, dtype=jnp.int32)
    cols = jnp.arange(2 * WIN, dtype=jnp.int32)
    bd3 = (cols[None, None, :] ==
           (2 * jnp.arange(WIN, dtype=jnp.int32)[:, None, None]
            + j2[None, :, None] // LATENT)).astype(jnp.float32)
    bdp = (j2[:, None] // LATENT ==
           jnp.arange(2, dtype=jnp.int32)[None, :]).astype(jnp.float32)
    t2 = (cols[None, :] % 2 ==
          jnp.arange(2, dtype=jnp.int32)[:, None]).astype(jnp.float32)

    def blkdiag(a):
        m, n = a.shape
        z = jnp.zeros((m, n), a.dtype)
        return jnp.concatenate(
            [jnp.concatenate([a, z], axis=1),
             jnp.concatenate([z, a], axis=1)], axis=0)

    wfc = blkdiag(W_f[:D])
    wfx = blkdiag(W_f[D:])
    wmu2 = blkdiag(W_mu)
    wsig2 = blkdiag(W_sig.reshape(H, 1))
    bf2 = jnp.concatenate([b_f.reshape(1, H)] * 2, axis=1)
    bmu2 = jnp.concatenate([b_mu.reshape(1, LATENT)] * 2, axis=1)
    bsig2 = jnp.concatenate([b_sig.reshape(1, 1)] * 2, axis=1)

    Bb = 512
    tc = _build_tc(B, WIN, D, LATENT, H, Bb)
    kl_sum, hinge_sum = tc(
        enc_c, enc_ctx.reshape(WIN, B // 2, 2 * D), mu_c,
        mu_ctx.reshape(WIN, B // 2, 2 * LATENT),
        mu_neg.reshape(WIN, B // 2, 2 * LATENT),
        ls_c.reshape(2, B // 2).transpose(1, 0), il40(ls_ctx), il40(ls_neg),
        wfc, wfx, bf2, wmu2, bmu2, wsig2, bsig2, bdp, bd3, t2)

    kl = kl_sum[0, 0] / B
    max_margin = hinge_sum[0, 0] / (B * WIN)
    return (kl, max_margin)
